# baseline (speedup 1.0000x reference)
import functools

import jax
import jax.numpy as jnp
from jax import lax
from jax.experimental import pallas as pl
from jax.experimental.pallas import tpu as pltpu
from jax.experimental.pallas import tpu_sc as plsc


def _tc_body(nbatch, z_ref, w_ref, inds_ref, loss_ref, q_ref):
    z = z_ref[...]
    w = w_ref[...]
    ones_row = jnp.ones((1, w.shape[1]), jnp.float32)
    wsq = lax.dot_general(
        ones_row, w * w, (((1,), (1,)), ((), ())),
        precision=lax.Precision.HIGHEST,
        preferred_element_type=jnp.float32,
    )

    d = lax.dot_general(
        z, w, (((1,), (1,)), ((), ())),
        precision=lax.Precision.HIGHEST,
        preferred_element_type=jnp.float32,
    )
    score = wsq - 2.0 * d

    m = jnp.min(score, axis=1, keepdims=True)
    kiota = lax.broadcasted_iota(jnp.int32, score.shape, 1)
    inds = jnp.min(jnp.where(score <= m, kiota, score.shape[1]),
                   axis=1, keepdims=True)
    inds_ref[...] = inds

    dl = jnp.sum(jnp.where(kiota == inds, d, 0.0),
                 axis=1, keepdims=True)
    t_per_b = dl.shape[0] // nbatch
    acc = jnp.zeros((1, 1), jnp.float32)
    for b in range(nbatch):
        seg = dl[b * t_per_b:(b + 1) * t_per_b]
        mx = jnp.max(seg, axis=0, keepdims=True)
        e = jnp.exp(seg - mx)
        s = jnp.sum(e, axis=0, keepdims=True)
        acc = acc + jnp.sum(e / s, axis=0, keepdims=True)
    loss_ref[...] = acc[0, 0]

    onehot = (kiota == inds).astype(jnp.float32)
    q_ref[...] = lax.dot_general(
        onehot, w, (((1,), (0,)), ((), ())),
        precision=lax.Precision.HIGHEST,
        preferred_element_type=jnp.float32,
    )


def _sc_gather(table, idx):
    V, D = table.shape
    (N,) = idx.shape
    info = plsc.get_sparse_core_info()
    nw = info.num_cores * info.num_subcores
    assert N % (8 * nw) == 0 and D % info.num_lanes == 0
    n_per_w = N // nw
    mesh = plsc.VectorSubcoreMesh(core_axis_name="c", subcore_axis_name="s")

    @functools.partial(
        pl.kernel, mesh=mesh,
        out_type=jax.ShapeDtypeStruct((N, D), jnp.float32),
        compiler_params=pltpu.CompilerParams(use_tc_tiling_on_sc=False),
        scratch_types=[
            pltpu.VMEM((n_per_w,), jnp.int32),
            pltpu.VMEM((n_per_w, D), jnp.float32),
            pltpu.SemaphoreType.DMA,
        ],
    )
    def k(table_hbm, idx_hbm, out_hbm, idx_v, rows_v, sem):
        wid = lax.axis_index("s") * info.num_cores + lax.axis_index("c")
        base = wid * n_per_w
        pltpu.sync_copy(idx_hbm.at[pl.ds(base, n_per_w)], idx_v)
        pltpu.async_copy(table_hbm.at[idx_v], rows_v, sem).wait()
        pltpu.sync_copy(rows_v, out_hbm.at[pl.ds(base, n_per_w)])

    return k(table, idx)


def kernel(latents, weight):
    B, T, D = latents.shape
    N = B * T

    inds2, loss, q = pl.pallas_call(
        functools.partial(_tc_body, B),
        out_specs=[
            pl.BlockSpec(memory_space=pltpu.VMEM),
            pl.BlockSpec(memory_space=pltpu.SMEM),
            pl.BlockSpec(memory_space=pltpu.VMEM),
        ],
        out_shape=[
            jax.ShapeDtypeStruct((N, 1), jnp.int32),
            jax.ShapeDtypeStruct((), jnp.float32),
            jax.ShapeDtypeStruct((N, D), jnp.float32),
        ],
    )(latents.reshape(N, D), weight)

    return (q.reshape(B, T, D), loss)


# speedup vs baseline: 1.0320x; 1.0320x over previous
import functools

import jax
import jax.numpy as jnp
from jax import lax
from jax.experimental import pallas as pl
from jax.experimental.pallas import tpu as pltpu
from jax.experimental.pallas import tpu_sc as plsc


def _tc_body(nbatch, z_ref, w_ref, inds_ref, loss_ref):
    z = z_ref[...]
    w = w_ref[...]
    ones_row = jnp.ones((1, w.shape[1]), jnp.float32)
    wsq = lax.dot_general(
        ones_row, w * w, (((1,), (1,)), ((), ())),
        precision=lax.Precision.HIGHEST,
        preferred_element_type=jnp.float32,
    )

    d = lax.dot_general(
        z, w, (((1,), (1,)), ((), ())),
        precision=lax.Precision.HIGHEST,
        preferred_element_type=jnp.float32,
    )
    score = wsq - 2.0 * d

    m = jnp.min(score, axis=1, keepdims=True)
    kiota = lax.broadcasted_iota(jnp.int32, score.shape, 1)
    inds = jnp.min(jnp.where(score <= m, kiota, score.shape[1]),
                   axis=1, keepdims=True)
    inds_ref[...] = inds

    dl = jnp.sum(jnp.where(kiota == inds, d, 0.0),
                 axis=1, keepdims=True)
    t_per_b = dl.shape[0] // nbatch
    acc = jnp.zeros((1, 1), jnp.float32)
    for b in range(nbatch):
        seg = dl[b * t_per_b:(b + 1) * t_per_b]
        mx = jnp.max(seg, axis=0, keepdims=True)
        e = jnp.exp(seg - mx)
        s = jnp.sum(e, axis=0, keepdims=True)
        acc = acc + jnp.sum(e / s, axis=0, keepdims=True)
    loss_ref[...] = acc[0, 0]


def _sc_gather(table, idx):
    V, D = table.shape
    (N,) = idx.shape
    info = plsc.get_sparse_core_info()
    nw = info.num_cores * info.num_subcores
    assert N % (8 * nw) == 0 and D % info.num_lanes == 0
    n_per_w = N // nw
    mesh = plsc.VectorSubcoreMesh(core_axis_name="c", subcore_axis_name="s")

    @functools.partial(
        pl.kernel, mesh=mesh,
        out_type=jax.ShapeDtypeStruct((N, D), jnp.float32),
        compiler_params=pltpu.CompilerParams(use_tc_tiling_on_sc=False),
        scratch_types=[
            pltpu.VMEM((n_per_w,), jnp.int32),
            pltpu.VMEM((n_per_w, D), jnp.float32),
            pltpu.SemaphoreType.DMA,
        ],
    )
    def k(table_hbm, idx_hbm, out_hbm, idx_v, rows_v, sem):
        wid = lax.axis_index("s") * info.num_cores + lax.axis_index("c")
        base = wid * n_per_w
        pltpu.sync_copy(idx_hbm.at[pl.ds(base, n_per_w)], idx_v)
        pltpu.async_copy(table_hbm.at[idx_v], rows_v, sem).wait()
        pltpu.sync_copy(rows_v, out_hbm.at[pl.ds(base, n_per_w)])

    return k(table, idx)


def kernel(latents, weight):
    B, T, D = latents.shape
    N = B * T

    inds2, loss = pl.pallas_call(
        functools.partial(_tc_body, B),
        out_specs=[
            pl.BlockSpec(memory_space=pltpu.VMEM),
            pl.BlockSpec(memory_space=pltpu.SMEM),
        ],
        out_shape=[
            jax.ShapeDtypeStruct((N, 1), jnp.int32),
            jax.ShapeDtypeStruct((), jnp.float32),
        ],
    )(latents.reshape(N, D), weight)

    q = _sc_gather(weight, inds2.reshape(N))
    return (q.reshape(B, T, D), loss)


# speedup vs baseline: 1.1270x; 1.0921x over previous
import functools

import jax
import jax.numpy as jnp
from jax import lax
from jax.experimental import pallas as pl
from jax.experimental.pallas import tpu as pltpu
from jax.experimental.pallas import tpu_sc as plsc


def _tc_body(nbatch, z_ref, w_ref, inds_ref, loss_ref):
    z3 = z_ref[...]
    z = z3.reshape(-1, z3.shape[-1])
    w = w_ref[...]
    n = z.shape[0]
    k = w.shape[0]

    wsq2 = 0.5 * jnp.sum(w * w, axis=1, keepdims=True)
    dT = lax.dot_general(
        w, z, (((1,), (1,)), ((), ())),
        precision=lax.Precision.HIGHEST,
        preferred_element_type=jnp.float32,
    )
    score = wsq2 - dT

    m = jnp.min(score, axis=0, keepdims=True)
    kiota = lax.broadcasted_iota(jnp.int32, score.shape, 0)
    inds = jnp.min(jnp.where(score <= m, kiota, k),
                   axis=0, keepdims=True)
    inds_ref[...] = inds.reshape(n)

    dl = jnp.sum(jnp.where(kiota == inds, dT, 0.0),
                 axis=0, keepdims=True)
    t_per_b = n // nbatch
    tio = lax.broadcasted_iota(jnp.int32, dl.shape, 1)
    acc = jnp.zeros((1, 1), jnp.float32)
    for b in range(nbatch):
        mask = (tio >= b * t_per_b) & (tio < (b + 1) * t_per_b)
        mx = jnp.max(jnp.where(mask, dl, -1e30), axis=1, keepdims=True)
        e = jnp.exp(jnp.where(mask, dl - mx, -1e30))
        s = jnp.sum(e, axis=1, keepdims=True)
        acc = acc + jnp.sum(e / s, axis=1, keepdims=True)
    loss_ref[...] = acc[0, 0]


def _sc_gather(table, idx):
    V, D = table.shape
    (N,) = idx.shape
    info = plsc.get_sparse_core_info()
    nw = info.num_cores * info.num_subcores
    assert N % (8 * nw) == 0 and D % info.num_lanes == 0
    n_per_w = N // nw
    mesh = plsc.VectorSubcoreMesh(core_axis_name="c", subcore_axis_name="s")

    @functools.partial(
        pl.kernel, mesh=mesh,
        out_type=jax.ShapeDtypeStruct((N, D), jnp.float32),
        compiler_params=pltpu.CompilerParams(use_tc_tiling_on_sc=False),
        scratch_types=[
            pltpu.VMEM((n_per_w,), jnp.int32),
            pltpu.VMEM((n_per_w, D), jnp.float32),
            pltpu.SemaphoreType.DMA,
        ],
    )
    def k(table_hbm, idx_hbm, out_hbm, idx_v, rows_v, sem):
        wid = lax.axis_index("s") * info.num_cores + lax.axis_index("c")
        base = wid * n_per_w
        pltpu.sync_copy(idx_hbm.at[pl.ds(base, n_per_w)], idx_v)
        pltpu.async_copy(table_hbm.at[idx_v], rows_v, sem).wait()
        pltpu.sync_copy(rows_v, out_hbm.at[pl.ds(base, n_per_w)])

    return k(table, idx)


def kernel(latents, weight):
    B, T, D = latents.shape
    N = B * T

    inds1, loss = pl.pallas_call(
        functools.partial(_tc_body, B),
        out_specs=[
            pl.BlockSpec(memory_space=pltpu.VMEM),
            pl.BlockSpec(memory_space=pltpu.SMEM),
        ],
        out_shape=[
            jax.ShapeDtypeStruct((N,), jnp.int32),
            jax.ShapeDtypeStruct((), jnp.float32),
        ],
    )(latents, weight)

    q = _sc_gather(weight, inds1)
    return (q.reshape(B, T, D), loss)


# speedup vs baseline: 1.1374x; 1.0092x over previous
import functools

import jax
import jax.numpy as jnp
from jax import lax
from jax.experimental import pallas as pl
from jax.experimental.pallas import tpu as pltpu
from jax.experimental.pallas import tpu_sc as plsc


def _tc_body(segs_per_step, t_per_b, z_ref, w_ref, inds_ref, loss_ref):
    z = z_ref[...]
    w = w_ref[...]
    nb = z.shape[0]
    k = w.shape[0]

    wsq2 = 0.5 * jnp.sum(w * w, axis=1, keepdims=True)
    dT = lax.dot_general(
        w, z, (((1,), (1,)), ((), ())),
        precision=lax.Precision.HIGHEST,
        preferred_element_type=jnp.float32,
    )
    score = wsq2 - dT

    m = jnp.min(score, axis=0, keepdims=True)
    kiota = lax.broadcasted_iota(jnp.int32, score.shape, 0)
    inds = jnp.min(jnp.where(score <= m, kiota, k),
                   axis=0, keepdims=True)
    off = pl.multiple_of(pl.program_id(0) * nb, 128)
    inds_ref[pl.ds(off, nb)] = inds.reshape(nb)

    dl = jnp.sum(jnp.where(kiota == inds, dT, 0.0),
                 axis=0, keepdims=True)
    tio = lax.broadcasted_iota(jnp.int32, dl.shape, 1)
    acc = jnp.zeros((1, 1), jnp.float32)
    for b in range(segs_per_step):
        mask = (tio >= b * t_per_b) & (tio < (b + 1) * t_per_b)
        mx = jnp.max(jnp.where(mask, dl, -1e30), axis=1, keepdims=True)
        e = jnp.exp(jnp.where(mask, dl - mx, -1e30))
        s = jnp.sum(e, axis=1, keepdims=True)
        acc = acc + jnp.sum(e / s, axis=1, keepdims=True)

    @pl.when(pl.program_id(0) == 0)
    def _():
        loss_ref[0] = 0.0

    loss_ref[0] += acc[0, 0]


def _sc_gather(table, idx):
    V, D = table.shape
    (N,) = idx.shape
    info = plsc.get_sparse_core_info()
    nw = info.num_cores * info.num_subcores
    assert N % (8 * nw) == 0 and D % info.num_lanes == 0
    n_per_w = N // nw
    mesh = plsc.VectorSubcoreMesh(core_axis_name="c", subcore_axis_name="s")

    @functools.partial(
        pl.kernel, mesh=mesh,
        out_type=jax.ShapeDtypeStruct((N, D), jnp.float32),
        compiler_params=pltpu.CompilerParams(use_tc_tiling_on_sc=False),
        scratch_types=[
            pltpu.VMEM((n_per_w,), jnp.int32),
            pltpu.VMEM((n_per_w, D), jnp.float32),
            pltpu.SemaphoreType.DMA,
        ],
    )
    def k(table_hbm, idx_hbm, out_hbm, idx_v, rows_v, sem):
        wid = lax.axis_index("s") * info.num_cores + lax.axis_index("c")
        base = wid * n_per_w
        pltpu.sync_copy(idx_hbm.at[pl.ds(base, n_per_w)], idx_v)
        pltpu.async_copy(table_hbm.at[idx_v], rows_v, sem).wait()
        pltpu.sync_copy(rows_v, out_hbm.at[pl.ds(base, n_per_w)])

    return k(table, idx)


def kernel(latents, weight):
    B, T, D = latents.shape
    K = weight.shape[0]
    N = B * T
    steps = 2
    nb = N // steps
    segs = B // steps

    inds1, loss = pl.pallas_call(
        functools.partial(_tc_body, segs, T),
        grid=(steps,),
        in_specs=[
            pl.BlockSpec((nb, D), lambda i: (i, 0)),
            pl.BlockSpec((K, D), lambda i: (0, 0)),
        ],
        out_specs=[
            pl.BlockSpec((N,), lambda i: (0,)),
            pl.BlockSpec((1,), lambda i: (0,), memory_space=pltpu.SMEM),
        ],
        out_shape=[
            jax.ShapeDtypeStruct((N,), jnp.int32),
            jax.ShapeDtypeStruct((1,), jnp.float32),
        ],
    )(latents.reshape(N, D), weight)

    q = _sc_gather(weight, inds1)
    return (q.reshape(B, T, D), loss.reshape(()))


# speedup vs baseline: 1.1765x; 1.0344x over previous
import functools

import jax
import jax.numpy as jnp
from jax import lax
from jax.experimental import pallas as pl
from jax.experimental.pallas import tpu as pltpu
from jax.experimental.pallas import tpu_sc as plsc


def _tc_body(segs_per_step, t_per_b, z_ref, w_ref, inds_ref, loss_ref):
    z = z_ref[...]
    w = w_ref[...]
    nb = z.shape[0]
    k = w.shape[0]

    wsq2 = 0.5 * jnp.sum(w * w, axis=1, keepdims=True)
    dT = lax.dot_general(
        w, z, (((1,), (1,)), ((), ())),
        precision=lax.Precision.HIGHEST,
        preferred_element_type=jnp.float32,
    )
    score = wsq2 - dT

    m = jnp.min(score, axis=0, keepdims=True)
    kiota = lax.broadcasted_iota(jnp.int32, score.shape, 0)
    inds = jnp.min(jnp.where(score <= m, kiota, k),
                   axis=0, keepdims=True)
    off = pl.multiple_of(pl.program_id(0) * nb, 128)
    inds_ref[pl.ds(off, nb)] = inds.reshape(nb)

    dl = jnp.sum(jnp.where(kiota == inds, dT, 0.0),
                 axis=0, keepdims=True)
    tio = lax.broadcasted_iota(jnp.int32, dl.shape, 1)
    acc = jnp.zeros((1, 1), jnp.float32)
    for b in range(segs_per_step):
        mask = (tio >= b * t_per_b) & (tio < (b + 1) * t_per_b)
        mx = jnp.max(jnp.where(mask, dl, -1e30), axis=1, keepdims=True)
        e = jnp.exp(jnp.where(mask, dl - mx, -1e30))
        s = jnp.sum(e, axis=1, keepdims=True)
        acc = acc + jnp.sum(e / s, axis=1, keepdims=True)

    @pl.when(pl.program_id(0) == 0)
    def _():
        loss_ref[0] = 0.0

    loss_ref[0] += acc[0, 0]


def _sc_gather(table, idx):
    V, D = table.shape
    (N,) = idx.shape
    info = plsc.get_sparse_core_info()
    nw = info.num_subcores
    assert N % (8 * nw) == 0 and D % info.num_lanes == 0
    n_per_w = N // nw
    mesh = plsc.VectorSubcoreMesh(core_axis_name="c", subcore_axis_name="s",
                                  num_cores=1)

    @functools.partial(
        pl.kernel, mesh=mesh,
        out_type=jax.ShapeDtypeStruct((N, D), jnp.float32),
        compiler_params=pltpu.CompilerParams(use_tc_tiling_on_sc=False),
        scratch_types=[
            pltpu.VMEM((n_per_w,), jnp.int32),
            pltpu.VMEM((n_per_w, D), jnp.float32),
            pltpu.SemaphoreType.DMA,
        ],
    )
    def k(table_hbm, idx_hbm, out_hbm, idx_v, rows_v, sem):
        wid = lax.axis_index("s")
        base = wid * n_per_w
        pltpu.sync_copy(idx_hbm.at[pl.ds(base, n_per_w)], idx_v)
        pltpu.async_copy(table_hbm.at[idx_v], rows_v, sem).wait()
        pltpu.sync_copy(rows_v, out_hbm.at[pl.ds(base, n_per_w)])

    return k(table, idx)


def kernel(latents, weight):
    B, T, D = latents.shape
    K = weight.shape[0]
    N = B * T
    steps = 2
    nb = N // steps
    segs = B // steps

    inds1, loss = pl.pallas_call(
        functools.partial(_tc_body, segs, T),
        grid=(steps,),
        in_specs=[
            pl.BlockSpec((nb, D), lambda i: (i, 0)),
            pl.BlockSpec((K, D), lambda i: (0, 0)),
        ],
        out_specs=[
            pl.BlockSpec((N,), lambda i: (0,)),
            pl.BlockSpec((1,), lambda i: (0,), memory_space=pltpu.SMEM),
        ],
        out_shape=[
            jax.ShapeDtypeStruct((N,), jnp.int32),
            jax.ShapeDtypeStruct((1,), jnp.float32),
        ],
    )(latents.reshape(N, D), weight)

    q = _sc_gather(weight, inds1)
    return (q.reshape(B, T, D), loss.reshape(()))


# speedup vs baseline: 1.1780x; 1.0012x over previous
import functools

import jax
import jax.numpy as jnp
from jax import lax
from jax.experimental import pallas as pl
from jax.experimental.pallas import tpu as pltpu
from jax.experimental.pallas import tpu_sc as plsc


def _tc_body(segs_per_step, t_per_b, z_ref, w_ref, inds_ref, loss_ref):
    z = z_ref[...]
    w = w_ref[...]
    nb = z.shape[0]
    k = w.shape[0]

    wsq2 = 0.5 * jnp.sum(w * w, axis=1, keepdims=True)
    dT = lax.dot_general(
        w, z, (((1,), (1,)), ((), ())),
        precision=lax.Precision.HIGHEST,
        preferred_element_type=jnp.float32,
    )
    score = wsq2 - dT

    m = jnp.min(score, axis=0, keepdims=True)
    kiota = lax.broadcasted_iota(jnp.int32, score.shape, 0)
    inds = jnp.min(jnp.where(score <= m, kiota, k),
                   axis=0, keepdims=True)
    off = pl.multiple_of(pl.program_id(0) * nb, 128)
    inds_ref[pl.ds(off, nb)] = inds.reshape(nb)

    dl = jnp.sum(jnp.where(kiota == inds, dT, 0.0),
                 axis=0, keepdims=True)
    tio = lax.broadcasted_iota(jnp.int32, dl.shape, 1)
    acc = jnp.zeros((1, 1), jnp.float32)
    for b in range(segs_per_step):
        mask = (tio >= b * t_per_b) & (tio < (b + 1) * t_per_b)
        mx = jnp.max(jnp.where(mask, dl, -1e30), axis=1, keepdims=True)
        e = jnp.exp(jnp.where(mask, dl - mx, -1e30))
        s = jnp.sum(e, axis=1, keepdims=True)
        acc = acc + jnp.sum(e / s, axis=1, keepdims=True)

    @pl.when(pl.program_id(0) == 0)
    def _():
        loss_ref[0] = 0.0

    loss_ref[0] += acc[0, 0]


def _sc_gather(table, idx):
    V, D = table.shape
    (N,) = idx.shape
    info = plsc.get_sparse_core_info()
    nw = info.num_subcores
    assert N % (8 * nw) == 0 and D % info.num_lanes == 0
    n_per_w = N // nw
    mesh = plsc.VectorSubcoreMesh(core_axis_name="c", subcore_axis_name="s",
                                  num_cores=1)

    @functools.partial(
        pl.kernel, mesh=mesh,
        out_type=jax.ShapeDtypeStruct((N, D), jnp.float32),
        compiler_params=pltpu.CompilerParams(use_tc_tiling_on_sc=False),
        scratch_types=[
            pltpu.VMEM((n_per_w,), jnp.int32),
            pltpu.VMEM((n_per_w // 2, D), jnp.float32),
            pltpu.VMEM((n_per_w // 2, D), jnp.float32),
            pltpu.SemaphoreType.DMA,
            pltpu.SemaphoreType.DMA,
            pltpu.SemaphoreType.DMA,
            pltpu.SemaphoreType.DMA,
        ],
    )
    def k(table_hbm, idx_hbm, out_hbm, idx_v, rows_a, rows_b,
          sa, sb, swa, swb):
        wid = lax.axis_index("s")
        base = wid * n_per_w
        h = n_per_w // 2
        pltpu.sync_copy(idx_hbm.at[pl.ds(base, n_per_w)], idx_v)
        ga = pltpu.async_copy(table_hbm.at[idx_v.at[pl.ds(0, h)]], rows_a, sa)
        gb = pltpu.async_copy(table_hbm.at[idx_v.at[pl.ds(h, h)]], rows_b, sb)
        ga.wait()
        wa = pltpu.async_copy(rows_a, out_hbm.at[pl.ds(base, h)], swa)
        gb.wait()
        wb = pltpu.async_copy(rows_b, out_hbm.at[pl.ds(base + h, h)], swb)
        wa.wait()
        wb.wait()

    return k(table, idx)


def kernel(latents, weight):
    B, T, D = latents.shape
    K = weight.shape[0]
    N = B * T
    steps = 2
    nb = N // steps
    segs = B // steps

    inds1, loss = pl.pallas_call(
        functools.partial(_tc_body, segs, T),
        grid=(steps,),
        in_specs=[
            pl.BlockSpec((nb, D), lambda i: (i, 0)),
            pl.BlockSpec((K, D), lambda i: (0, 0)),
        ],
        out_specs=[
            pl.BlockSpec((N,), lambda i: (0,)),
            pl.BlockSpec((1,), lambda i: (0,), memory_space=pltpu.SMEM),
        ],
        out_shape=[
            jax.ShapeDtypeStruct((N,), jnp.int32),
            jax.ShapeDtypeStruct((1,), jnp.float32),
        ],
    )(latents.reshape(N, D), weight)

    q = _sc_gather(weight, inds1)
    return (q.reshape(B, T, D), loss.reshape(()))
